# baseline (speedup 1.0000x reference)
import functools

import jax
import jax.numpy as jnp
from jax import lax
from jax.experimental import pallas as pl
from jax.experimental.pallas import tpu as pltpu
from jax.experimental.pallas import tpu_sc as plsc

EMBED = 64
CHUNK = 128


@functools.lru_cache(maxsize=None)
def _build_gather(n_workers: int, n_chunks: int, vocab: int):
    mesh = plsc.VectorSubcoreMesh(core_axis_name="c", subcore_axis_name="s")
    per_w = n_chunks * CHUNK

    @functools.partial(
        pl.kernel,
        mesh=mesh,
        out_type=jax.ShapeDtypeStruct((n_workers * per_w, EMBED), jnp.float32),
        scratch_types=[
            pltpu.VMEM((n_chunks, CHUNK), jnp.int32),
            pltpu.VMEM((CHUNK, EMBED), jnp.float32),
            pltpu.SemaphoreType.DMA,
        ],
        compiler_params=pltpu.CompilerParams(use_tc_tiling_on_sc=False),
    )
    def gather_kernel(table_hbm, idx_hbm, out_hbm, idx_v, rows_v, sem):
        nc = lax.axis_size("c")
        wid = lax.axis_index("s") * nc + lax.axis_index("c")
        base = wid * per_w
        pltpu.sync_copy(idx_hbm.at[wid], idx_v)

        def step(j, carry):
            pltpu.async_copy(table_hbm.at[idx_v.at[j]], rows_v, sem).wait()
            pltpu.sync_copy(rows_v, out_hbm.at[pl.ds(base + j * CHUNK, CHUNK)])
            return carry

        lax.fori_loop(0, n_chunks, step, 0)

    return gather_kernel


def kernel(x, table):
    b, s = x.shape
    vocab, embed = table.shape
    assert embed == EMBED
    total = b * s
    n_workers = 32
    assert total % (n_workers * CHUNK) == 0
    n_chunks = total // (n_workers * CHUNK)
    idx = x.reshape(n_workers, n_chunks, CHUNK).astype(jnp.int32)
    out = _build_gather(n_workers, n_chunks, vocab)(table, idx)
    return out.reshape(b, s, embed)


# speedup vs baseline: 1.1160x; 1.1160x over previous
import functools

import jax
import jax.numpy as jnp
from jax import lax
from jax.experimental import pallas as pl
from jax.experimental.pallas import tpu as pltpu
from jax.experimental.pallas import tpu_sc as plsc

EMBED = 64
CHUNK = 128


K = 4


@functools.lru_cache(maxsize=None)
def _build_gather(n_workers: int, n_chunks: int, vocab: int):
    mesh = plsc.VectorSubcoreMesh(core_axis_name="c", subcore_axis_name="s")
    per_w = n_chunks * CHUNK
    assert n_chunks % K == 0
    n_blocks = n_chunks // K
    blk = K * CHUNK

    @functools.partial(
        pl.kernel,
        mesh=mesh,
        out_type=jax.ShapeDtypeStruct((n_workers * per_w, EMBED), jnp.float32),
        scratch_types=[
            pltpu.VMEM((n_chunks, CHUNK), jnp.int32),
            pltpu.VMEM((2, blk, EMBED), jnp.float32),
            pltpu.SemaphoreType.DMA,
            pltpu.SemaphoreType.DMA,
        ],
        compiler_params=pltpu.CompilerParams(use_tc_tiling_on_sc=False),
    )
    def gather_kernel(table_hbm, idx_hbm, out_hbm, idx_v, rows_v, gsem, osem):
        nc = lax.axis_size("c")
        wid = lax.axis_index("s") * nc + lax.axis_index("c")
        base = wid * per_w
        pltpu.sync_copy(idx_hbm.at[wid], idx_v)

        def fire(t, b):
            for k in range(K):
                pltpu.make_async_copy(
                    table_hbm.at[idx_v.at[t * K + k]],
                    rows_v.at[b, pl.ds(k * CHUNK, CHUNK)],
                    gsem,
                ).start()

        def wait_gathers(b):
            for k in range(K):
                pltpu.make_async_copy(
                    table_hbm.at[idx_v.at[k]],
                    rows_v.at[b, pl.ds(k * CHUNK, CHUNK)],
                    gsem,
                ).wait()

        def out_desc(t, b):
            return pltpu.make_async_copy(
                rows_v.at[b], out_hbm.at[pl.ds(base + t * blk, blk)], osem
            )

        fire(0, 0)

        def step(t, carry):
            b = lax.rem(t, 2)
            pl.when(t >= 1)(lambda: out_desc(t - 1, 1 - b).wait())
            pl.when(t < n_blocks - 1)(lambda: fire(t + 1, 1 - b))
            wait_gathers(b)
            out_desc(t, b).start()
            return carry

        lax.fori_loop(0, n_blocks, step, 0)
        out_desc(n_blocks - 1, lax.rem(n_blocks - 1, 2)).wait()

    return gather_kernel


def kernel(x, table):
    b, s = x.shape
    vocab, embed = table.shape
    assert embed == EMBED
    total = b * s
    n_workers = 32
    assert total % (n_workers * CHUNK) == 0
    n_chunks = total // (n_workers * CHUNK)
    idx = x.reshape(n_workers, n_chunks, CHUNK).astype(jnp.int32)
    out = _build_gather(n_workers, n_chunks, vocab)(table, idx)
    return out.reshape(b, s, embed)


# speedup vs baseline: 1.1478x; 1.0285x over previous
import functools

import jax
import jax.numpy as jnp
from jax import lax
from jax.experimental import pallas as pl
from jax.experimental.pallas import tpu as pltpu
from jax.experimental.pallas import tpu_sc as plsc

EMBED = 64
CHUNK = 128
K = 4


@functools.lru_cache(maxsize=None)
def _build_gather(seq: int, batch: int, vocab: int):
    mesh = plsc.VectorSubcoreMesh(core_axis_name="c", subcore_axis_name="s")
    n_workers = 32
    assert batch % (n_workers * CHUNK) == 0 or batch == n_workers * CHUNK
    n_chunks = seq
    assert n_chunks % K == 0
    n_blocks = n_chunks // K
    blk = K * CHUNK

    @functools.partial(
        pl.kernel,
        mesh=mesh,
        out_type=jax.ShapeDtypeStruct((seq * batch, EMBED), jnp.float32),
        scratch_types=[
            pltpu.VMEM((n_chunks, CHUNK), jnp.int32),
            pltpu.VMEM((2, blk, EMBED), jnp.float32),
            pltpu.SemaphoreType.DMA,
            pltpu.SemaphoreType.DMA,
        ],
        compiler_params=pltpu.CompilerParams(use_tc_tiling_on_sc=False),
    )
    def gather_kernel(table_hbm, idx_hbm, out_hbm, idx_v, rows_v, gsem, osem):
        nc = lax.axis_size("c")
        wid = lax.axis_index("s") * nc + lax.axis_index("c")
        col0 = wid * CHUNK
        pltpu.sync_copy(idx_hbm.at[:, pl.ds(col0, CHUNK)], idx_v)

        def fire(t, b):
            for k in range(K):
                pltpu.make_async_copy(
                    table_hbm.at[idx_v.at[t * K + k]],
                    rows_v.at[b, pl.ds(k * CHUNK, CHUNK)],
                    gsem,
                ).start()

        def wait_gathers(b):
            for k in range(K):
                pltpu.make_async_copy(
                    table_hbm.at[idx_v.at[k]],
                    rows_v.at[b, pl.ds(k * CHUNK, CHUNK)],
                    gsem,
                ).wait()

        def out_copies(t, b):
            return [
                pltpu.make_async_copy(
                    rows_v.at[b, pl.ds(k * CHUNK, CHUNK)],
                    out_hbm.at[pl.ds((t * K + k) * batch + col0, CHUNK)],
                    osem,
                )
                for k in range(K)
            ]

        fire(0, 0)

        def step(t, carry):
            b = lax.rem(t, 2)
            def drain_prev():
                for c in out_copies(t - 1, 1 - b):
                    c.wait()

            pl.when(t >= 1)(drain_prev)
            pl.when(t < n_blocks - 1)(lambda: fire(t + 1, 1 - b))
            wait_gathers(b)
            for c in out_copies(t, b):
                c.start()
            return carry

        lax.fori_loop(0, n_blocks, step, 0)
        for c in out_copies(n_blocks - 1, (n_blocks - 1) % 2):
            c.wait()

    return gather_kernel


def kernel(x, table):
    b, s = x.shape
    vocab, embed = table.shape
    assert embed == EMBED
    xt = jnp.swapaxes(x, 0, 1).astype(jnp.int32)
    out = _build_gather(s, b, vocab)(table, xt)
    return jnp.swapaxes(out.reshape(s, b, embed), 0, 1)


# speedup vs baseline: 1.5333x; 1.3358x over previous
import functools

import jax
import jax.numpy as jnp
from jax import lax
from jax.experimental import pallas as pl
from jax.experimental.pallas import tpu as pltpu
from jax.experimental.pallas import tpu_sc as plsc

EMBED = 64
CHUNK = 128
K = 4


@functools.lru_cache(maxsize=None)
def _build_gather(seq: int, batch: int, vocab: int):
    mesh = plsc.VectorSubcoreMesh(core_axis_name="c", subcore_axis_name="s")
    n_workers = 32
    assert batch % (n_workers * CHUNK) == 0 or batch == n_workers * CHUNK
    n_chunks = seq
    assert n_chunks % K == 0
    n_blocks = n_chunks // K
    blk = K * CHUNK

    @functools.partial(
        pl.kernel,
        mesh=mesh,
        out_type=jax.ShapeDtypeStruct((seq * batch, 2 * EMBED), jnp.float32),
        scratch_types=[
            pltpu.VMEM((n_chunks, CHUNK), jnp.int32),
            pltpu.VMEM((2, blk, EMBED), jnp.float32),
            pltpu.SemaphoreType.DMA,
            pltpu.SemaphoreType.DMA,
        ],
        compiler_params=pltpu.CompilerParams(use_tc_tiling_on_sc=False),
    )
    def gather_kernel(table_hbm, idx_hbm, out_hbm, idx_v, rows_v, gsem, osem):
        nc = lax.axis_size("c")
        wid = lax.axis_index("s") * nc + lax.axis_index("c")
        col0 = wid * CHUNK
        pltpu.sync_copy(idx_hbm.at[:, pl.ds(col0, CHUNK)], idx_v)

        def fire(t, b):
            for k in range(K):
                pltpu.make_async_copy(
                    table_hbm.at[idx_v.at[t * K + k]],
                    rows_v.at[b, pl.ds(k * CHUNK, CHUNK)],
                    gsem,
                ).start()

        def wait_gathers(b):
            for k in range(K):
                pltpu.make_async_copy(
                    table_hbm.at[idx_v.at[k]],
                    rows_v.at[b, pl.ds(k * CHUNK, CHUNK)],
                    gsem,
                ).wait()

        def out_copies(t, b):
            return [
                pltpu.make_async_copy(
                    rows_v.at[b, pl.ds(k * CHUNK, CHUNK)],
                    out_hbm.at[
                        pl.ds((t * K + k) * batch + col0, CHUNK),
                        pl.ds(0, EMBED),
                    ],
                    osem,
                )
                for k in range(K)
            ]

        fire(0, 0)

        def step(t, carry):
            b = lax.rem(t, 2)
            def drain_prev():
                for c in out_copies(t - 1, 1 - b):
                    c.wait()

            pl.when(t >= 1)(drain_prev)
            pl.when(t < n_blocks - 1)(lambda: fire(t + 1, 1 - b))
            wait_gathers(b)
            for c in out_copies(t, b):
                c.start()
            return carry

        lax.fori_loop(0, n_blocks, step, 0)
        for c in out_copies(n_blocks - 1, (n_blocks - 1) % 2):
            c.wait()

    return gather_kernel


def kernel(x, table):
    b, s = x.shape
    vocab, embed = table.shape
    assert embed == EMBED
    xt = jnp.swapaxes(x, 0, 1).astype(jnp.int32)
    out = _build_gather(s, b, vocab)(table, xt)
    return jnp.swapaxes(out[:, :embed].reshape(s, b, embed), 0, 1)
